# baseline (speedup 1.0000x reference)
import functools

import jax
import jax.numpy as jnp
from jax.experimental import pallas as pl
from jax.experimental.pallas import tpu as pltpu

B, C, H, W = 4, 32, 128, 128
K = 32
F = 32
T = 16384
NP = H * W
NT = NP // T


def _stats_kernel(tr_ref, cnt_in_ref, feat_in_ref, var_in_ref, l_ref, e_ref,
                  cnt_out_ref, feat_out_ref, var_out_ref,
                  c_acc, s_acc, m_acc):
    bi = pl.program_id(0)
    ti = pl.program_id(1)

    @pl.when((bi == 0) & (ti == 0))
    def _init():
        c_acc[...] = jnp.zeros_like(c_acc)
        s_acc[...] = jnp.zeros_like(s_acc)
        m_acc[...] = jnp.zeros_like(m_acc)

    TH = T // 8
    riota = jax.lax.broadcasted_iota(jnp.int32, (C, TH), 0)
    aio = jax.lax.broadcasted_iota(jnp.int32, (4, TH), 0)
    bio = jax.lax.broadcasted_iota(jnp.int32, (8, TH), 0)
    c_parts, s_parts, m_parts = [], [], []
    for h in range(8):
        sl = pl.ds(h * TH, TH)
        lg = l_ref[0, :, sl]
        xb = e_ref[0, :, sl]

        lu = jax.lax.bitcast_convert_type(lg, jnp.int32)
        enc = jax.lax.bitcast_convert_type((lu & ~31) | (31 - riota),
                                           jnp.float32)
        mx = jnp.max(enc, axis=0, keepdims=True)
        g = 31 - (jax.lax.bitcast_convert_type(mx, jnp.int32) & 31)
        onehot = (enc == mx).astype(jnp.float32)

        c_part = onehot[:, 0:128]
        for j in range(1, TH // 128):
            c_part = c_part + onehot[:, j * 128:(j + 1) * 128]
        c_parts.append(c_part)

        s_parts.append(jax.lax.dot_general(
            onehot.astype(jnp.bfloat16), xb.astype(jnp.bfloat16),
            (((1,), (1,)), ((), ())),
            preferred_element_type=jnp.float32))

        xb16 = xb.astype(jnp.bfloat16)
        a_m = (aio == (g >> 3)).astype(jnp.bfloat16)
        b_m = (bio == (g & 7)).astype(jnp.bfloat16)
        w1 = jnp.concatenate([xb16 * a_m[a:a + 1, :] for a in range(4)],
                             axis=0)
        w2 = jnp.concatenate([xb16 * b_m[b:b + 1, :] for b in range(8)],
                             axis=0)
        m_parts.append(jax.lax.dot_general(
            w1, w2, (((1,), (1,)), ((), ())),
            preferred_element_type=jnp.float32))

    c_acc[...] += sum(c_parts)
    s_acc[...] += sum(s_parts)
    m_acc[...] += sum(m_parts)

    @pl.when((bi == B - 1) & (ti == NT - 1))
    def _finalize():
        n_k = jnp.sum(c_acc[...], axis=1, keepdims=True)
        denom = cnt_in_ref[...].reshape(K, 1) + n_k
        s = s_acc[...]
        f_new = feat_in_ref[...] + s / denom
        m4 = m_acc[...].reshape(4, F, 8, F)
        m3 = jnp.transpose(m4, (0, 2, 1, 3)).reshape(K, F, F)
        cross = (f_new[:, :, None] * s[:, None, :]
                 + s[:, :, None] * f_new[:, None, :])
        sq = n_k[:, :, None] * f_new[:, :, None] * f_new[:, None, :]
        cov = m3 - cross + sq
        train = tr_ref[...] != 0
        var_out_ref[...] = jnp.where(
            train.reshape(1, 1, 1),
            var_in_ref[...] + cov / denom[:, :, None], var_in_ref[...])
        feat_out_ref[...] = jnp.where(train, f_new, feat_in_ref[...])
        cnt_out_ref[...] = jnp.where(
            train, cnt_in_ref[...] + n_k.reshape(1, K), cnt_in_ref[...])


@functools.partial(jax.jit, static_argnames=())
def _run_stats(train_flag, logits_r, emb_r, count_in, features, var):
    grid = (B, NT)
    kernel_fn = pl.pallas_call(
        _stats_kernel,
        grid=grid,
        in_specs=[
            pl.BlockSpec((1, 1), lambda b, t: (0, 0)),
            pl.BlockSpec((1, K), lambda b, t: (0, 0)),
            pl.BlockSpec((K, F), lambda b, t: (0, 0)),
            pl.BlockSpec((K, F, F), lambda b, t: (0, 0, 0)),
            pl.BlockSpec((1, C, T), lambda b, t: (b, 0, t)),
            pl.BlockSpec((1, F, T), lambda b, t: (b, 0, t)),
        ],
        out_specs=[
            pl.BlockSpec((1, K), lambda b, t: (0, 0)),
            pl.BlockSpec((K, F), lambda b, t: (0, 0)),
            pl.BlockSpec((K, F, F), lambda b, t: (0, 0, 0)),
        ],
        out_shape=[
            jax.ShapeDtypeStruct((1, K), jnp.float32),
            jax.ShapeDtypeStruct((K, F), jnp.float32),
            jax.ShapeDtypeStruct((K, F, F), jnp.float32),
        ],
        scratch_shapes=[
            pltpu.VMEM((K, 128), jnp.float32),
            pltpu.VMEM((K, F), jnp.float32),
            pltpu.VMEM((4 * F, 8 * F), jnp.float32),
        ],
        compiler_params=pltpu.CompilerParams(
            dimension_semantics=("arbitrary", "arbitrary"),
        ),
    )
    return kernel_fn(train_flag, count_in.reshape(1, K), features, var,
                     logits_r, emb_r)


def kernel(unified_embedding, logits, gt, is_train, dataset_ids, count,
           features, var):
    logits_r = logits.reshape(B, C, NP)
    emb_r = unified_embedding.reshape(B, F, NP)
    train_flag = jnp.asarray(is_train, jnp.int32).reshape(1, 1)
    cnt_new, feat_new, var_new = _run_stats(train_flag, logits_r, emb_r,
                                            count, features, var)
    acc_loss = jnp.float32(0.0)
    return (acc_loss, feat_new, var_new, cnt_new.reshape(K))


# speedup vs baseline: 1.0130x; 1.0130x over previous
import functools

import jax
import jax.numpy as jnp
from jax.experimental import pallas as pl
from jax.experimental.pallas import tpu as pltpu

B, C, H, W = 4, 32, 128, 128
K = 32
F = 32
T = 8192
NP = H * W
NT = NP // T


def _stats_kernel(tr_ref, cnt_in_ref, feat_in_ref, var_in_ref, l_ref, e_ref,
                  cnt_out_ref, feat_out_ref, var_out_ref,
                  c_acc, s_acc, m_acc):
    bi = pl.program_id(0)
    ti = pl.program_id(1)

    @pl.when((bi == 0) & (ti == 0))
    def _init():
        c_acc[...] = jnp.zeros_like(c_acc)
        s_acc[...] = jnp.zeros_like(s_acc)
        m_acc[...] = jnp.zeros_like(m_acc)

    TH = T // 4
    riota = jax.lax.broadcasted_iota(jnp.int32, (C, TH), 0)
    aio = jax.lax.broadcasted_iota(jnp.int32, (4, TH), 0)
    bio = jax.lax.broadcasted_iota(jnp.int32, (8, TH), 0)
    c_parts, s_parts, m_parts = [], [], []
    for h in range(4):
        sl = pl.ds(h * TH, TH)
        lg = l_ref[0, :, sl]
        xb = e_ref[0, :, sl]

        lu = jax.lax.bitcast_convert_type(lg, jnp.int32)
        enc = jax.lax.bitcast_convert_type((lu & ~31) | (31 - riota),
                                           jnp.float32)
        mx = jnp.max(enc, axis=0, keepdims=True)
        g = 31 - (jax.lax.bitcast_convert_type(mx, jnp.int32) & 31)
        onehot = (enc == mx).astype(jnp.float32)

        c_part = onehot[:, 0:128]
        for j in range(1, TH // 128):
            c_part = c_part + onehot[:, j * 128:(j + 1) * 128]
        c_parts.append(c_part)

        s_parts.append(jax.lax.dot_general(
            onehot.astype(jnp.bfloat16), xb.astype(jnp.bfloat16),
            (((1,), (1,)), ((), ())),
            preferred_element_type=jnp.float32))

        xb16 = xb.astype(jnp.bfloat16)
        a_m = (aio == (g >> 3)).astype(jnp.bfloat16)
        b_m = (bio == (g & 7)).astype(jnp.bfloat16)
        w1 = jnp.concatenate([xb16 * a_m[a:a + 1, :] for a in range(4)],
                             axis=0)
        w2 = jnp.concatenate([xb16 * b_m[b:b + 1, :] for b in range(8)],
                             axis=0)
        m_parts.append(jax.lax.dot_general(
            w1, w2, (((1,), (1,)), ((), ())),
            preferred_element_type=jnp.float32))

    c_acc[...] += sum(c_parts)
    s_acc[...] += sum(s_parts)
    m_acc[...] += sum(m_parts)

    @pl.when((bi == B - 1) & (ti == NT - 1))
    def _finalize():
        n_k = jnp.sum(c_acc[...], axis=1, keepdims=True)
        denom = cnt_in_ref[...].reshape(K, 1) + n_k
        s = s_acc[...]
        f_new = feat_in_ref[...] + s / denom
        m4 = m_acc[...].reshape(4, F, 8, F)
        m3 = jnp.transpose(m4, (0, 2, 1, 3)).reshape(K, F, F)
        cross = (f_new[:, :, None] * s[:, None, :]
                 + s[:, :, None] * f_new[:, None, :])
        sq = n_k[:, :, None] * f_new[:, :, None] * f_new[:, None, :]
        cov = m3 - cross + sq
        train = tr_ref[...] != 0
        var_out_ref[...] = jnp.where(
            train.reshape(1, 1, 1),
            var_in_ref[...] + cov / denom[:, :, None], var_in_ref[...])
        feat_out_ref[...] = jnp.where(train, f_new, feat_in_ref[...])
        cnt_out_ref[...] = jnp.where(
            train, cnt_in_ref[...] + n_k.reshape(1, K), cnt_in_ref[...])


@functools.partial(jax.jit, static_argnames=())
def _run_stats(train_flag, logits_r, emb_r, count_in, features, var):
    grid = (B, NT)
    kernel_fn = pl.pallas_call(
        _stats_kernel,
        grid=grid,
        in_specs=[
            pl.BlockSpec((1, 1), lambda b, t: (0, 0)),
            pl.BlockSpec((1, K), lambda b, t: (0, 0)),
            pl.BlockSpec((K, F), lambda b, t: (0, 0)),
            pl.BlockSpec((K, F, F), lambda b, t: (0, 0, 0)),
            pl.BlockSpec((1, C, T), lambda b, t: (b, 0, t)),
            pl.BlockSpec((1, F, T), lambda b, t: (b, 0, t)),
        ],
        out_specs=[
            pl.BlockSpec((1, K), lambda b, t: (0, 0)),
            pl.BlockSpec((K, F), lambda b, t: (0, 0)),
            pl.BlockSpec((K, F, F), lambda b, t: (0, 0, 0)),
        ],
        out_shape=[
            jax.ShapeDtypeStruct((1, K), jnp.float32),
            jax.ShapeDtypeStruct((K, F), jnp.float32),
            jax.ShapeDtypeStruct((K, F, F), jnp.float32),
        ],
        scratch_shapes=[
            pltpu.VMEM((K, 128), jnp.float32),
            pltpu.VMEM((K, F), jnp.float32),
            pltpu.VMEM((4 * F, 8 * F), jnp.float32),
        ],
        compiler_params=pltpu.CompilerParams(
            dimension_semantics=("arbitrary", "arbitrary"),
        ),
    )
    return kernel_fn(train_flag, count_in.reshape(1, K), features, var,
                     logits_r, emb_r)


def kernel(unified_embedding, logits, gt, is_train, dataset_ids, count,
           features, var):
    logits_r = logits.reshape(B, C, NP)
    emb_r = unified_embedding.reshape(B, F, NP)
    train_flag = jnp.asarray(is_train, jnp.int32).reshape(1, 1)
    cnt_new, feat_new, var_new = _run_stats(train_flag, logits_r, emb_r,
                                            count, features, var)
    acc_loss = jnp.float32(0.0)
    return (acc_loss, feat_new, var_new, cnt_new.reshape(K))
